# baseline (device time: 30277 ns/iter reference)
import functools

import jax
import jax.numpy as jnp
from jax import lax
from jax.experimental import pallas as pl
from jax.experimental.pallas import tpu as pltpu

T_PER = 256
D = 512
F = 1024
E_LOCAL = 2


def _top2_weights(g):
    v1 = jnp.max(g, axis=1, keepdims=True)
    t1 = g >= v1
    gm = jnp.where(t1, -jnp.inf, g)
    v2 = jnp.max(gm, axis=1, keepdims=True)
    t2 = gm >= v2
    w1 = 1.0 / (1.0 + jnp.exp(v2 - v1))
    return jnp.where(t1, w1, 0.0) + jnp.where(t2, 1.0 - w1, 0.0)


def kernel(x, router, W1, W2):
    def body(x_ref, r_ref, w1_ref, w2_ref, out_ref,
             xr_ref, rr_ref, pn_ref, pr_ref, send_sems, recv_sems):
        my_x = lax.axis_index("x")
        my_y = lax.axis_index("y")
        nbr = (my_x, 1 - my_y)

        barrier_sem = pltpu.get_barrier_semaphore()
        pl.semaphore_signal(barrier_sem, inc=1, device_id=nbr,
                            device_id_type=pl.DeviceIdType.MESH)
        pl.semaphore_wait(barrier_sem, 1)

        rdma_x = pltpu.make_async_remote_copy(
            src_ref=x_ref, dst_ref=xr_ref,
            send_sem=send_sems.at[0], recv_sem=recv_sems.at[0],
            device_id=nbr, device_id_type=pl.DeviceIdType.MESH)
        rdma_x.start()
        rdma_r = pltpu.make_async_remote_copy(
            src_ref=r_ref, dst_ref=rr_ref,
            send_sem=send_sems.at[1], recv_sem=recv_sems.at[1],
            device_id=nbr, device_id_type=pl.DeviceIdType.MESH)
        rdma_r.start()
        rdma_x.wait()
        rdma_r.wait()

        x_my = x_ref[...]
        x_nb = xr_ref[...]

        def gates(xx):
            return jnp.concatenate(
                [jnp.dot(xx, r_ref[...], preferred_element_type=jnp.float32),
                 jnp.dot(xx, rr_ref[...], preferred_element_type=jnp.float32)],
                axis=1)

        wt_my = _top2_weights(gates(x_my))
        wt_nb = _top2_weights(gates(x_nb))

        xb_my = x_my.astype(jnp.bfloat16)
        xb_nb = x_nb.astype(jnp.bfloat16)
        acc_my = jnp.zeros((T_PER, D), jnp.float32)
        acc_nb = jnp.zeros((T_PER, D), jnp.float32)
        for le in range(E_LOCAL):
            w1e = w1_ref[le].astype(jnp.bfloat16)
            w2e = w2_ref[le].astype(jnp.bfloat16)

            def expert(xb):
                h = jnp.dot(xb, w1e, preferred_element_type=jnp.float32)
                h = jnp.maximum(h, 0.0).astype(jnp.bfloat16)
                return jnp.dot(h, w2e, preferred_element_type=jnp.float32)

            acc_my = acc_my + expert(xb_my) * wt_my[:, le:le + 1]
            acc_nb = acc_nb + expert(xb_nb) * wt_nb[:, le:le + 1]

        pn_ref[...] = acc_nb
        rdma_p = pltpu.make_async_remote_copy(
            src_ref=pn_ref, dst_ref=pr_ref,
            send_sem=send_sems.at[2], recv_sem=recv_sems.at[2],
            device_id=nbr, device_id_type=pl.DeviceIdType.MESH)
        rdma_p.start()
        rdma_p.wait()

        out_ref[...] = acc_my + pr_ref[...]

    return pl.pallas_call(
        body,
        out_shape=jax.ShapeDtypeStruct((T_PER, D), jnp.float32),
        in_specs=[pl.BlockSpec(memory_space=pltpu.VMEM)] * 4,
        out_specs=pl.BlockSpec(memory_space=pltpu.VMEM),
        scratch_shapes=[
            pltpu.VMEM((T_PER, D), jnp.float32),
            pltpu.VMEM((D, 2), jnp.float32),
            pltpu.VMEM((T_PER, D), jnp.float32),
            pltpu.VMEM((T_PER, D), jnp.float32),
            pltpu.SemaphoreType.DMA((3,)),
            pltpu.SemaphoreType.DMA((3,)),
        ],
        compiler_params=pltpu.CompilerParams(collective_id=0),
    )(x, router, W1, W2)


# device time: 26203 ns/iter; 1.1555x vs baseline; 1.1555x over previous
import jax
import jax.numpy as jnp
from jax import lax
from jax.experimental import pallas as pl
from jax.experimental.pallas import tpu as pltpu

T_PER = 256
D = 512
F = 1024
E_LOCAL = 2


def _top2_weights(g):
    v1 = jnp.max(g, axis=1, keepdims=True)
    t1 = g >= v1
    gm = jnp.where(t1, -jnp.inf, g)
    v2 = jnp.max(gm, axis=1, keepdims=True)
    t2 = gm >= v2
    w1 = 1.0 / (1.0 + jnp.exp(v2 - v1))
    return jnp.where(t1, w1, 0.0) + jnp.where(t2, 1.0 - w1, 0.0)


def kernel(x, router, W1, W2):
    def body(x_ref, r_ref, w1_ref, w2_ref, out_ref,
             xb_my_ref, xb_nb_ref, rr_ref, wts_ref, wtr_ref,
             pn_ref, pr_ref, send_sems, recv_sems):
        my_x = lax.axis_index("x")
        my_y = lax.axis_index("y")
        nbr = (my_x, 1 - my_y)

        def rdma(i, src, dst):
            return pltpu.make_async_remote_copy(
                src_ref=src, dst_ref=dst,
                send_sem=send_sems.at[i], recv_sem=recv_sems.at[i],
                device_id=nbr, device_id_type=pl.DeviceIdType.MESH)

        barrier_sem = pltpu.get_barrier_semaphore()
        pl.semaphore_signal(barrier_sem, inc=1, device_id=nbr,
                            device_id_type=pl.DeviceIdType.MESH)
        pl.semaphore_wait(barrier_sem, 1)

        xb_my_ref[...] = x_ref[...].astype(jnp.bfloat16)
        rdma_x = rdma(0, xb_my_ref, xb_nb_ref)
        rdma_x.start()
        rdma_r = rdma(1, r_ref, rr_ref)
        rdma_r.start()

        w1b = [w1_ref[le].astype(jnp.bfloat16) for le in range(E_LOCAL)]
        w2b = [w2_ref[le].astype(jnp.bfloat16) for le in range(E_LOCAL)]

        def expert(xb, le):
            h = jnp.dot(xb, w1b[le], preferred_element_type=jnp.float32)
            h = jnp.maximum(h, 0.0).astype(jnp.bfloat16)
            return jnp.dot(h, w2b[le], preferred_element_type=jnp.float32)

        xbm = xb_my_ref[...]
        o_my = [expert(xbm, le) for le in range(E_LOCAL)]

        rdma_r.wait()
        x_my = x_ref[...]
        g = jnp.concatenate(
            [jnp.dot(x_my, r_ref[...], preferred_element_type=jnp.float32),
             jnp.dot(x_my, rr_ref[...], preferred_element_type=jnp.float32)],
            axis=1)
        wt = _top2_weights(g)

        wts_ref[...] = wt[:, E_LOCAL:]
        rdma_w = rdma(2, wts_ref, wtr_ref)
        rdma_w.start()

        acc_my = o_my[0] * wt[:, 0:1] + o_my[1] * wt[:, 1:2]

        rdma_x.wait()
        rdma_w.wait()
        xbn = xb_nb_ref[...]
        wr = wtr_ref[...]
        acc_nb = (expert(xbn, 0) * wr[:, 0:1]
                  + expert(xbn, 1) * wr[:, 1:2])

        pn_ref[...] = acc_nb.astype(jnp.bfloat16)
        rdma_p = rdma(3, pn_ref, pr_ref)
        rdma_p.start()
        rdma_p.wait()

        out_ref[...] = acc_my + pr_ref[...].astype(jnp.float32)

    return pl.pallas_call(
        body,
        out_shape=jax.ShapeDtypeStruct((T_PER, D), jnp.float32),
        in_specs=[pl.BlockSpec(memory_space=pltpu.VMEM)] * 4,
        out_specs=pl.BlockSpec(memory_space=pltpu.VMEM),
        scratch_shapes=[
            pltpu.VMEM((T_PER, D), jnp.bfloat16),
            pltpu.VMEM((T_PER, D), jnp.bfloat16),
            pltpu.VMEM((D, E_LOCAL), jnp.float32),
            pltpu.VMEM((T_PER, E_LOCAL), jnp.float32),
            pltpu.VMEM((T_PER, E_LOCAL), jnp.float32),
            pltpu.VMEM((T_PER, D), jnp.bfloat16),
            pltpu.VMEM((T_PER, D), jnp.bfloat16),
            pltpu.SemaphoreType.DMA((4,)),
            pltpu.SemaphoreType.DMA((4,)),
        ],
        compiler_params=pltpu.CompilerParams(collective_id=0),
    )(x, router, W1, W2)
